# baseline (speedup 1.0000x reference)
import functools

import jax
import jax.numpy as jnp
from jax import lax
from jax.experimental import pallas as pl
from jax.experimental.pallas import tpu as pltpu
from jax.experimental.pallas import tpu_sc as plsc

NF = 26
VOCAB = 100000
DIM = 16
BATCH = 16384
PACK = 128 // DIM
LANES = 16

CH_COLS = 1408
NCH = 71
EPR = 16
CH_ROWS = CH_COLS // EPR
NP = NCH * CH_ROWS + 8

_INFO = plsc.get_sparse_core_info()
_NC = _INFO.num_cores
_NS = _INFO.num_subcores
_NW = _NC * _NS
_BPW = BATCH // _NW
_SUB = 128
_NSUB = _BPW // _SUB


def _pack_body(*refs):
    wt_refs = refs[:NF]
    tail_ref = refs[NF]
    out_ref = refs[NF + 1]
    lbuf = refs[NF + 2]
    stag = refs[NF + 3]
    lsem0, lsem1, wsem0, wsem1 = refs[NF + 4:NF + 8]

    wid = lax.axis_index("s") * _NC + lax.axis_index("c")
    iota = lax.iota(jnp.int32, LANES)
    ccol = [iota * 8 + w for w in range(8)]

    def load_of(f, c, par, sem):
        col0 = c * CH_COLS
        return (
            pltpu.make_async_copy(
                wt_refs[f].at[pl.ds(0, 8), pl.ds(col0, CH_COLS)],
                lbuf.at[par, pl.ds(0, 8)], sem),
            pltpu.make_async_copy(
                wt_refs[f].at[pl.ds(8, 8), pl.ds(col0, CH_COLS)],
                lbuf.at[par, pl.ds(8, 8)], sem),
        )

    def write_of(f, c, par, sem):
        return pltpu.make_async_copy(
            stag.at[par], out_ref.at[f, pl.ds(c * CH_ROWS, CH_ROWS)], sem)

    def base_of(f):
        return lax.rem(wid + (7 * f) % 32, _NW)

    for cp in load_of(0, base_of(0), 0, lsem0):
        cp.start()

    for f in range(NF):
        base_c = base_of(f)
        has3 = base_c < NCH - 2 * _NW

        def do_rep(rep, _, f=f, base_c=base_c, has3=has3):
            c = base_c + rep * _NW
            active = jnp.logical_or(rep < 2, has3)

            @pl.when(active)
            def _():
                par = lax.rem(rep, 2)
                nxt_active = jnp.logical_or(rep == 0,
                                            jnp.logical_and(rep == 1, has3))

                @pl.when(jnp.logical_and(nxt_active, par == 0))
                def _():
                    for cp in load_of(f, c + _NW, 1, lsem1):
                        cp.start()

                @pl.when(jnp.logical_and(nxt_active, par == 1))
                def _():
                    for cp in load_of(f, c + _NW, 0, lsem0):
                        cp.start()

                @pl.when(par == 0)
                def _():
                    for cp in load_of(f, c, 0, lsem0):
                        cp.wait()

                @pl.when(par == 1)
                def _():
                    for cp in load_of(f, c, 1, lsem1):
                        cp.wait()

                @pl.when(jnp.logical_and(rep >= 2, par == 0))
                def _():
                    write_of(f, c - 2 * _NW, 0, wsem0).wait()

                def do_group(g, _, par=par):
                    r_vec = iota * 0 + g
                    for w in range(8):
                        a = lbuf[par, 2 * w, pl.ds(g * LANES, LANES)]
                        b = lbuf[par, 2 * w + 1, pl.ds(g * LANES, LANES)]
                        packed = plsc.pack(a, b,
                                           format=plsc.PackFormat.INTERLEAVED)
                        pi32 = plsc.bitcast(packed, jnp.int32)
                        plsc.store_scatter(stag.at[par], [r_vec, ccol[w]],
                                           pi32)
                    return ()

                lax.fori_loop(0, CH_COLS // LANES, do_group, (),
                              unroll=2)

                @pl.when(par == 0)
                def _():
                    write_of(f, c, 0, wsem0).start()

                @pl.when(par == 1)
                def _():
                    write_of(f, c, 1, wsem1).start()

            return ()

        lax.fori_loop(0, 3, do_rep, (), unroll=False)

        if f + 1 < NF:
            for cp in load_of(f + 1, base_of(f + 1), 0, lsem0):
                cp.start()

        @pl.when(jnp.logical_not(has3))
        def _():
            write_of(f, base_c, 0, wsem0).wait()

        write_of(f, base_c + _NW, 1, wsem1).wait()

        @pl.when(has3)
        def _():
            write_of(f, base_c + 2 * _NW, 0, wsem0).wait()

    @pl.when(wid == 0)
    def _():
        for f in range(NF):
            pltpu.sync_copy(tail_ref.at[f], out_ref.at[f, pl.ds(NP - 8, 8)])


def _gather_body(idx_hbm, wlin_hbm, out_hbm,
                 idx_all, rows_v, stag,
                 idx_sem, gat_sem0, gat_sem1, out_sem0, out_sem1):
    wid = lax.axis_index("s") * _NC + lax.axis_index("c")
    iota = lax.iota(jnp.int32, LANES)
    wvecs = [jnp.full((LANES,), w, jnp.int32) for w in range(8)]

    def do_subchunk(sb, _):
        base = wid * _BPW + sb * _SUB
        tc = base // _SUB

        pltpu.async_copy(idx_hbm.at[:, pl.ds(base, _SUB)], idx_all, idx_sem)
        pltpu.make_async_copy(idx_hbm.at[:, pl.ds(base, _SUB)], idx_all,
                              idx_sem).wait()

        def gather_of(f, par, sem):
            return pltpu.make_async_copy(
                wlin_hbm.at[f].at[idx_all.at[f]],
                rows_v.at[par], sem)

        def out_of(f, par, sem):
            return pltpu.make_async_copy(
                stag.at[par],
                out_hbm.at[pl.ds(2 * f, 2), tc],
                sem)

        gather_of(0, 0, gat_sem0).start()

        def do_field(f, _):
            par = lax.rem(f, 2)

            @pl.when(jnp.logical_and(f + 1 < NF, par == 0))
            def _():
                gather_of(f + 1, 1, gat_sem1).start()

            @pl.when(jnp.logical_and(f + 1 < NF, par == 1))
            def _():
                gather_of(f + 1, 0, gat_sem0).start()

            @pl.when(par == 0)
            def _():
                gather_of(f, 0, gat_sem0).wait()

            @pl.when(par == 1)
            def _():
                gather_of(f, 1, gat_sem1).wait()

            @pl.when(jnp.logical_and(f >= 2, par == 0))
            def _():
                out_of(f - 2, 0, out_sem0).wait()

            @pl.when(jnp.logical_and(f >= 2, par == 1))
            def _():
                out_of(f - 2, 1, out_sem1).wait()

            for g in range(_SUB // LANES):
                i_vec = iota + (g * LANES)
                for w in range(8):
                    words = plsc.load_gather(rows_v.at[par],
                                             [i_vec, wvecs[w]])
                    bf = plsc.bitcast(words, jnp.bfloat16)
                    a, b = plsc.unpack(bf,
                                       format=plsc.PackFormat.INTERLEAVED)
                    d0, d1 = 2 * w, 2 * w + 1
                    stag[par, d0 // 8, d0 % 8, pl.ds(g * LANES, LANES)] = a
                    stag[par, d1 // 8, d1 % 8, pl.ds(g * LANES, LANES)] = b

            @pl.when(par == 0)
            def _():
                out_of(f, 0, out_sem0).start()

            @pl.when(par == 1)
            def _():
                out_of(f, 1, out_sem1).start()

            return ()

        lax.fori_loop(0, NF, do_field, (), unroll=False)

        out_of(NF - 2, 0, out_sem0).wait()
        out_of(NF - 1, 1, out_sem1).wait()
        return ()

    lax.fori_loop(0, _NSUB, do_subchunk, (), unroll=False)


@jax.jit
def _cat_embed(idx_list, tab_list):
    idxstack = jnp.stack(idx_list)

    wt_list = [w.T for w in tab_list]
    tail = jnp.stack([
        lax.bitcast_convert_type(
            jnp.pad(w[VOCAB - 32:], ((0, 96), (0, 0)))
            .astype(jnp.bfloat16).reshape(8, 128, 2),
            jnp.int32)
        for w in tab_list])

    mesh = plsc.VectorSubcoreMesh(core_axis_name="c", subcore_axis_name="s")
    params = pltpu.CompilerParams(needs_layout_passes=False)

    pack = pl.kernel(
        _pack_body,
        mesh=mesh,
        out_type=jax.ShapeDtypeStruct((NF, NP, 128), jnp.int32),
        compiler_params=params,
        scratch_types=[
            pltpu.VMEM((2, DIM, CH_COLS), jnp.float32),
            pltpu.VMEM((2, CH_ROWS, 128), jnp.int32),
            pltpu.SemaphoreType.DMA,
            pltpu.SemaphoreType.DMA,
            pltpu.SemaphoreType.DMA,
            pltpu.SemaphoreType.DMA,
        ],
    )
    wstack = pack(*wt_list, tail)

    wlin = wstack.reshape(NF, NP * EPR, 8)

    gather = pl.kernel(
        _gather_body,
        mesh=mesh,
        out_type=jax.ShapeDtypeStruct((NF * DIM // 8, BATCH // _SUB, 8, _SUB),
                                      jnp.float32),
        compiler_params=pltpu.CompilerParams(
            needs_layout_passes=False, use_tc_tiling_on_sc=False),
        scratch_types=[
            pltpu.VMEM((NF, _SUB), jnp.int32),
            pltpu.VMEM((2, _SUB, 8), jnp.int32),
            pltpu.VMEM((2, 2, 8, _SUB), jnp.float32),
            pltpu.SemaphoreType.DMA,
            pltpu.SemaphoreType.DMA,
            pltpu.SemaphoreType.DMA,
            pltpu.SemaphoreType.DMA,
            pltpu.SemaphoreType.DMA,
        ],
    )
    out4 = gather(idxstack, wlin)
    return out4.transpose(1, 3, 0, 2).reshape(BATCH, NF * DIM)


def kernel(f00, f01, f02, f03, f04, f05, f06, f07, f08, f09,
           f10, f11, f12, f13, f14, f15, f16, f17, f18, f19,
           f20, f21, f22, f23, f24, f25,
           W_f00, W_f01, W_f02, W_f03, W_f04, W_f05, W_f06, W_f07,
           W_f08, W_f09, W_f10, W_f11, W_f12, W_f13, W_f14, W_f15,
           W_f16, W_f17, W_f18, W_f19, W_f20, W_f21, W_f22, W_f23,
           W_f24, W_f25):
    idx = [f00, f01, f02, f03, f04, f05, f06, f07, f08, f09,
           f10, f11, f12, f13, f14, f15, f16, f17, f18, f19,
           f20, f21, f22, f23, f24, f25]
    tabs = [W_f00, W_f01, W_f02, W_f03, W_f04, W_f05, W_f06, W_f07,
            W_f08, W_f09, W_f10, W_f11, W_f12, W_f13, W_f14, W_f15,
            W_f16, W_f17, W_f18, W_f19, W_f20, W_f21, W_f22, W_f23,
            W_f24, W_f25]
    idx = [i.astype(jnp.int32) for i in idx]
    return _cat_embed(idx, tabs)


# speedup vs baseline: 18.0575x; 18.0575x over previous
import functools

import jax
import jax.numpy as jnp
from jax import lax
from jax.experimental import pallas as pl
from jax.experimental.pallas import tpu as pltpu
from jax.experimental.pallas import tpu_sc as plsc

NF = 26
VOCAB = 100000
DIM = 16
BATCH = 16384
PACK = 128 // DIM
LANES = 16

CH_COLS = 1408
NCH = 71
CH_ROWS = CH_COLS // PACK
NP = 12504

_INFO = plsc.get_sparse_core_info()
_NC = _INFO.num_cores
_NS = _INFO.num_subcores
_NW = _NC * _NS
_BPW = BATCH // _NW
_SUB = 128
_NSUB = _BPW // _SUB


def _pack_body(*refs):
    wt_refs = refs[:NF]
    tail_ref = refs[NF]
    out_ref = refs[NF + 1]
    lbuf = refs[NF + 2]
    stag = refs[NF + 3]
    lsem0, lsem1, wsem0, wsem1 = refs[NF + 4:NF + 8]

    wid = lax.axis_index("s") * _NC + lax.axis_index("c")
    iota = lax.iota(jnp.int32, LANES)
    half = iota >> 3
    ccol = [(iota & 7) * DIM + d for d in range(DIM)]

    def load_of(f, c, par, sem):
        col0 = c * CH_COLS
        return (
            pltpu.make_async_copy(
                wt_refs[f].at[pl.ds(0, 8), pl.ds(col0, CH_COLS)],
                lbuf.at[par, pl.ds(0, 8)], sem),
            pltpu.make_async_copy(
                wt_refs[f].at[pl.ds(8, 8), pl.ds(col0, CH_COLS)],
                lbuf.at[par, pl.ds(8, 8)], sem),
        )

    def write_of(f, c, par, sem):
        return pltpu.make_async_copy(
            stag.at[par], out_ref.at[f, pl.ds(c * CH_ROWS, CH_ROWS)], sem)

    def base_of(f):
        return lax.rem(wid + (7 * f) % 32, _NW)

    for cp in load_of(0, base_of(0), 0, lsem0):
        cp.start()

    for f in range(NF):
        base_c = base_of(f)
        has3 = base_c < NCH - 2 * _NW

        def do_rep(rep, _, f=f, base_c=base_c, has3=has3):
            c = base_c + rep * _NW
            active = jnp.logical_or(rep < 2, has3)

            @pl.when(active)
            def _():
                par = lax.rem(rep, 2)
                nxt_active = jnp.logical_or(rep == 0,
                                            jnp.logical_and(rep == 1, has3))

                @pl.when(jnp.logical_and(nxt_active, par == 0))
                def _():
                    for cp in load_of(f, c + _NW, 1, lsem1):
                        cp.start()

                @pl.when(jnp.logical_and(nxt_active, par == 1))
                def _():
                    for cp in load_of(f, c + _NW, 0, lsem0):
                        cp.start()

                @pl.when(par == 0)
                def _():
                    for cp in load_of(f, c, 0, lsem0):
                        cp.wait()

                @pl.when(par == 1)
                def _():
                    for cp in load_of(f, c, 1, lsem1):
                        cp.wait()

                @pl.when(jnp.logical_and(rep >= 2, par == 0))
                def _():
                    write_of(f, c - 2 * _NW, 0, wsem0).wait()

                def do_group(g, _, par=par):
                    r_vec = half + 2 * g
                    for d in range(DIM):
                        row = lbuf[par, d, pl.ds(g * LANES, LANES)]
                        plsc.store_scatter(stag.at[par], [r_vec, ccol[d]],
                                           row)
                    return ()

                lax.fori_loop(0, CH_COLS // LANES, do_group, (),
                              unroll=2)

                @pl.when(par == 0)
                def _():
                    write_of(f, c, 0, wsem0).start()

                @pl.when(par == 1)
                def _():
                    write_of(f, c, 1, wsem1).start()

            return ()

        lax.fori_loop(0, 3, do_rep, (), unroll=False)

        if f + 1 < NF:
            for cp in load_of(f + 1, base_of(f + 1), 0, lsem0):
                cp.start()

        @pl.when(jnp.logical_not(has3))
        def _():
            write_of(f, base_c, 0, wsem0).wait()

        write_of(f, base_c + _NW, 1, wsem1).wait()

        @pl.when(has3)
        def _():
            write_of(f, base_c + 2 * _NW, 0, wsem0).wait()

    @pl.when(wid == 0)
    def _():
        for f in range(NF):
            pltpu.sync_copy(tail_ref.at[f], out_ref.at[f, pl.ds(NP - 8, 8)])


def _gather_body(idx_hbm, wlin_hbm, out_hbm,
                 idx_all, rows_v, stag,
                 idx_sem, gat_sem0, gat_sem1, gat_sem2, gat_sem3,
                 out_sem0, out_sem1):
    wid = lax.axis_index("s") * _NC + lax.axis_index("c")
    iota = lax.iota(jnp.int32, LANES)
    dvecs = [jnp.full((LANES,), d, jnp.int32) for d in range(DIM)]
    gsems = (gat_sem0, gat_sem1, gat_sem2, gat_sem3)

    wbase = wid * _BPW
    pltpu.async_copy(idx_hbm.at[:, pl.ds(wbase, _BPW)], idx_all, idx_sem)
    pltpu.make_async_copy(idx_hbm.at[:, pl.ds(wbase, _BPW)], idx_all,
                          idx_sem).wait()

    def do_subchunk(sb, _):
        base = wbase + sb * _SUB
        tc = base // _SUB

        def gather_of(f, q):
            return pltpu.make_async_copy(
                wlin_hbm.at[f].at[idx_all.at[f, pl.ds(sb * _SUB, _SUB)]],
                rows_v.at[q], gsems[q])

        def out_of(f, par, sem):
            return pltpu.make_async_copy(
                stag.at[par],
                out_hbm.at[pl.ds(2 * f, 2), tc],
                sem)

        gather_of(0, 0).start()
        gather_of(1, 1).start()

        def do_field(f, _):
            q = lax.rem(f, 4)
            par = lax.rem(f, 2)

            for k in range(4):
                @pl.when(jnp.logical_and(f + 2 < NF, q == k))
                def _(k=k):
                    gather_of(f + 2, (k + 2) % 4).start()

            for k in range(4):
                @pl.when(q == k)
                def _(k=k):
                    gather_of(f, k).wait()

            @pl.when(jnp.logical_and(f >= 2, par == 0))
            def _():
                out_of(f - 2, 0, out_sem0).wait()

            @pl.when(jnp.logical_and(f >= 2, par == 1))
            def _():
                out_of(f - 2, 1, out_sem1).wait()

            for g in range(_SUB // LANES):
                i_vec = iota + (g * LANES)
                for d in range(DIM):
                    vals = plsc.load_gather(rows_v.at[q],
                                            [i_vec, dvecs[d]])
                    stag[par, d // 8, d % 8, pl.ds(g * LANES, LANES)] = vals

            @pl.when(par == 0)
            def _():
                out_of(f, 0, out_sem0).start()

            @pl.when(par == 1)
            def _():
                out_of(f, 1, out_sem1).start()

            return ()

        lax.fori_loop(0, NF, do_field, (), unroll=False)

        out_of(NF - 2, 0, out_sem0).wait()
        out_of(NF - 1, 1, out_sem1).wait()
        return ()

    lax.fori_loop(0, _NSUB, do_subchunk, (), unroll=False)


@jax.jit
def _cat_embed(idx_list, tab_list):
    idxstack = jnp.stack(idx_list)

    wt_list = [w.T for w in tab_list]
    tail = jnp.stack([
        jnp.pad(w[VOCAB - 32:], ((0, 32), (0, 0))).reshape(8, 128)
        for w in tab_list])

    mesh = plsc.VectorSubcoreMesh(core_axis_name="c", subcore_axis_name="s")
    params = pltpu.CompilerParams(needs_layout_passes=False)

    pack = pl.kernel(
        _pack_body,
        mesh=mesh,
        out_type=jax.ShapeDtypeStruct((NF, NP, PACK * DIM), jnp.float32),
        compiler_params=params,
        scratch_types=[
            pltpu.VMEM((2, DIM, CH_COLS), jnp.float32),
            pltpu.VMEM((2, CH_ROWS, PACK * DIM), jnp.float32),
            pltpu.SemaphoreType.DMA,
            pltpu.SemaphoreType.DMA,
            pltpu.SemaphoreType.DMA,
            pltpu.SemaphoreType.DMA,
        ],
    )
    wstack = pack(*wt_list, tail)

    wlin = wstack.reshape(NF, NP * PACK, DIM)

    gather = pl.kernel(
        _gather_body,
        mesh=mesh,
        out_type=jax.ShapeDtypeStruct((NF * DIM // 8, BATCH // _SUB, 8, _SUB),
                                      jnp.float32),
        compiler_params=pltpu.CompilerParams(
            needs_layout_passes=False, use_tc_tiling_on_sc=False),
        scratch_types=[
            pltpu.VMEM((NF, _BPW), jnp.int32),
            pltpu.VMEM((4, _SUB, DIM), jnp.float32),
            pltpu.VMEM((2, 2, 8, _SUB), jnp.float32),
            pltpu.SemaphoreType.DMA,
            pltpu.SemaphoreType.DMA,
            pltpu.SemaphoreType.DMA,
            pltpu.SemaphoreType.DMA,
            pltpu.SemaphoreType.DMA,
            pltpu.SemaphoreType.DMA,
            pltpu.SemaphoreType.DMA,
        ],
    )
    out4 = gather(idxstack, wlin)
    return out4.transpose(1, 3, 0, 2).reshape(BATCH, NF * DIM)


def kernel(f00, f01, f02, f03, f04, f05, f06, f07, f08, f09,
           f10, f11, f12, f13, f14, f15, f16, f17, f18, f19,
           f20, f21, f22, f23, f24, f25,
           W_f00, W_f01, W_f02, W_f03, W_f04, W_f05, W_f06, W_f07,
           W_f08, W_f09, W_f10, W_f11, W_f12, W_f13, W_f14, W_f15,
           W_f16, W_f17, W_f18, W_f19, W_f20, W_f21, W_f22, W_f23,
           W_f24, W_f25):
    idx = [f00, f01, f02, f03, f04, f05, f06, f07, f08, f09,
           f10, f11, f12, f13, f14, f15, f16, f17, f18, f19,
           f20, f21, f22, f23, f24, f25]
    tabs = [W_f00, W_f01, W_f02, W_f03, W_f04, W_f05, W_f06, W_f07,
            W_f08, W_f09, W_f10, W_f11, W_f12, W_f13, W_f14, W_f15,
            W_f16, W_f17, W_f18, W_f19, W_f20, W_f21, W_f22, W_f23,
            W_f24, W_f25]
    idx = [i.astype(jnp.int32) for i in idx]
    return _cat_embed(idx, tabs)
